# baseline (speedup 1.0000x reference)
import functools

import jax
import jax.numpy as jnp
from jax import lax
from jax.experimental import pallas as pl
from jax.experimental.pallas import tpu as pltpu
from jax.experimental.pallas import tpu_sc as plsc

_NC, _NS = 2, 16
_NW = _NC * _NS
_CHUNK = 128
_SS = 16
_SB = _SS * _CHUNK
_C = 13
_TW = 16
_TZ = 64
_BIG = 1 << 20
_L = 16
_MAGIC13 = 5042


def _encode_table_body(zs_ref, t_ref):
    zs_b = jnp.broadcast_to(zs_ref[0:1, :], (_TZ, _TW))
    zrow = lax.broadcasted_iota(jnp.int32, (_TZ, _TW), 0)
    lane = lax.broadcasted_iota(jnp.int32, (_TZ, _TW), 1)
    j = jnp.sum((zs_b < zrow).astype(jnp.int32), axis=1, keepdims=True)
    j = jnp.minimum(j, _C - 1)
    exact = jnp.sum((zs_b == zrow).astype(jnp.int32), axis=1, keepdims=True) > 0
    lo = jnp.maximum(j - 1, 0)
    zs_f = zs_b.astype(jnp.float32)
    zf = lax.broadcasted_iota(jnp.int32, (_TZ, 1), 0).astype(jnp.float32)
    zs_hi = jnp.sum(jnp.where(lane == j, zs_f, 0.0), axis=1, keepdims=True)
    zs_lo = jnp.sum(jnp.where(lane == lo, zs_f, 0.0), axis=1, keepdims=True)
    denom = jnp.maximum(zs_hi - zs_lo, 1.0)
    w_lo = (zs_hi - zf) / denom
    w_hi = (zf - zs_lo) / denom
    onehot = (lane == j).astype(jnp.float32)
    interp = jnp.where(lane == lo, w_lo, 0.0) + jnp.where(lane == j, w_hi, 0.0)
    t_ref[...] = jnp.where(exact, onehot, interp)


def _encode_table(zs_tc):
    return pl.pallas_call(
        _encode_table_body,
        out_shape=jax.ShapeDtypeStruct((_TZ, _TW), jnp.float32),
    )(zs_tc)


@functools.lru_cache(maxsize=None)
def _sc_encode(nsb_w):
    nsb = nsb_w * _NW
    npad = nsb * _SB
    mesh = plsc.VectorSubcoreMesh(core_axis_name="c", subcore_axis_name="s")

    @functools.partial(
        pl.kernel,
        out_type=jax.ShapeDtypeStruct((npad * _C,), jnp.float32),
        mesh=mesh,
        compiler_params=pltpu.CompilerParams(
            use_tc_tiling_on_sc=False, needs_layout_passes=False),
        scratch_types=(
            [pltpu.VMEM((_SS, _CHUNK), jnp.int32) for _ in range(2)]
            + [pltpu.VMEM((_SB, _TW), jnp.float32) for _ in range(2)]
            + [pltpu.VMEM((_SB * _C + 8,), jnp.float32) for _ in range(2)]
            + [pltpu.SemaphoreType.DMA] * 3
        ),
    )
    def enc(table_hbm, z_hbm, out_hbm, *rest):
        idx = rest[0:2]
        stg = rest[2:4]
        pkd = rest[4:6]
        isem, gsem, wsem = rest[6:9]
        w = lax.axis_index("s") * _NC + lax.axis_index("c")

        lane_i = lax.iota(jnp.int32, _L)
        packs = []
        for v in range(_C):
            i = lane_i + (v * _L)
            r = lax.shift_right_logical(i * _MAGIC13, 16)
            packs.append((r, i - r * _C))

        def sb_of(s):
            return w + _NW * s

        def idx_load(s):
            return pltpu.async_copy(
                z_hbm.at[pl.ds(sb_of(s) * _SS, _SS)], idx[s % 2], isem)

        def gathers(s):
            g = s % 2
            return [
                pltpu.async_copy(
                    table_hbm.at[idx[g].at[b]],
                    stg[g].at[pl.ds(b * _CHUNK, _CHUNK)],
                    gsem)
                for b in range(_SS)
            ]

        def pack(s):
            g = s % 2

            def body(grp, carry):
                rbase = grp * _L
                obase = grp * (_L * _C)
                for v, (rv, cv) in enumerate(packs):
                    vec = plsc.load_gather(stg[g], [rbase + rv, cv])
                    pkd[g][pl.ds(obase + v * _L, _L)] = vec
                return carry

            lax.fori_loop(0, _SB // _L, body, 0)

        def write(s):
            return pltpu.async_copy(
                pkd[s % 2].at[pl.ds(0, _SB * _C)],
                out_hbm.at[pl.ds(sb_of(s) * _SB * _C, _SB * _C)], wsem)

        il = idx_load(0)
        il.wait()
        gh = gathers(0)
        wh = [None, None]
        for s in range(nsb_w):
            nxt = idx_load(s + 1) if s + 1 < nsb_w else None
            for h in gh:
                h.wait()
            if nxt is not None:
                nxt.wait()
                gh = gathers(s + 1)
            if wh[s % 2] is not None:
                wh[s % 2].wait()
            pack(s)
            wh[s % 2] = write(s)
        wh[(nsb_w - 1) % 2].wait()
        wh[nsb_w % 2].wait()

    return enc


def kernel(atomic_numbers, zs):
    n = atomic_numbers.shape[0]
    C = zs.shape[0]
    nsb_w = -(-n // (_SB * _NW))
    npad = nsb_w * _NW * _SB
    z2d = jnp.pad(atomic_numbers.astype(jnp.int32),
                  (0, npad - n)).reshape(npad // _CHUNK, _CHUNK)
    zs_pad = jnp.pad(zs.astype(jnp.int32), (0, _TW - C), constant_values=_BIG)
    zs_tc = jnp.broadcast_to(zs_pad[None, :], (8, _TW))
    table = _encode_table(zs_tc)
    flat = _sc_encode(nsb_w)(table, z2d)
    return flat[:n * C].reshape(n, C)
